# baseline (speedup 1.0000x reference)
import functools

import jax
import jax.numpy as jnp
from jax import lax
from jax.experimental import pallas as pl
from jax.experimental.pallas import tpu as pltpu
from jax.experimental.pallas import tpu_sc as plsc

N = 10000
E = 320000
C_IN = 128
HID = 256
C_OUT = 128

NC = 2
NS = 16
NW = NC * NS

NPW = 320
NPAD = NW * NPW
RPT_LEN = NPW + 24
RPT_PAD = (NW - 1) * NPW + RPT_LEN
CH = 128
FB = 16
EPAD = E + CH

GA = -2.0 * 0.7978845608028654
GB = GA * 0.044715

VB = HID // 16


def _mm_ps_body(x_ref, w_ref, b1_ref, p_ref, s_ref):
    ps = jnp.dot(x_ref[...], w_ref[...], preferred_element_type=jnp.float32)
    p_ref[...] = ps[:, :HID]
    s_ref[...] = ps[:, HID:] + b1_ref[...]


def _mm_out_body(g_ref, w2_ref, b2_ref, rhi_ref, rlo_ref, o_ref):
    cnt = (rhi_ref[0, 0, :] - rlo_ref[0, 0, :]).astype(jnp.float32)
    scale = 1.0 / jnp.maximum(cnt, 1.0)
    gs = g_ref[...] * scale[:, None]
    y = jnp.dot(gs, w2_ref[...], preferred_element_type=jnp.float32)
    o_ref[...] = y + b2_ref[...] * (cnt > 0.0).astype(jnp.float32)[:, None]


def _sc_segment_gelu(p_hbm, s_hbm, idx_hbm, rpt_hbm, g_hbm,
                     rpt_v, idx_v, rows_v, s_all, flush_v, sem, sem2):
    c = lax.axis_index("c")
    s = lax.axis_index("s")
    wid = s * NC + c
    n0 = wid * NPW

    pltpu.async_copy(rpt_hbm.at[pl.ds(pl.multiple_of(n0, 8), RPT_LEN)],
                     rpt_v, sem2).wait()
    pltpu.async_copy(s_hbm.at[pl.ds(pl.multiple_of(n0, 8), NPW)],
                     s_all, sem2).wait()

    def rv(k):
        return rpt_v[pl.ds(k, 16)][0]

    zeros16 = jnp.zeros((16,), jnp.float32)

    def node_body(i, _):
        e0 = rv(i)
        e1 = rv(i + 1)
        srow = tuple(s_all[i, pl.ds(j * 16, 16)] for j in range(VB))
        a0 = (e0 // 8) * 8
        nch = (e1 - a0 + CH - 1) // CH

        def chunk_body(k, acc):
            a = a0 + k * CH
            pltpu.async_copy(idx_hbm.at[pl.ds(pl.multiple_of(a, 8), CH)],
                             idx_v, sem2).wait()
            pltpu.async_copy(p_hbm.at[idx_v], rows_v, sem).wait()
            lo = jnp.maximum(e0, a) - a
            hi = jnp.minimum(e1, a + CH) - a

            def edge_body(r, acc_):
                new_acc = []
                for j in range(VB):
                    x = rows_v[r, pl.ds(j * 16, 16)] + srow[j]
                    z = x * (GA + GB * (x * x))
                    new_acc.append(acc_[j] + x / (1.0 + jnp.exp(z)))
                return tuple(new_acc)

            return lax.fori_loop(lo, hi, edge_body, acc)

        acc = lax.fori_loop(0, nch, chunk_body, (zeros16,) * VB)
        for j in range(VB):
            flush_v[i % FB, pl.ds(j * 16, 16)] = acc[j]

        @pl.when(i % FB == FB - 1)
        def _():
            pltpu.async_copy(
                flush_v,
                g_hbm.at[pl.ds(pl.multiple_of(n0 + (i // FB) * FB, 8), FB)],
                sem2).wait()

        return 0

    lax.fori_loop(0, NPW, node_body, 0)


@functools.partial(
    pl.kernel,
    mesh=plsc.VectorSubcoreMesh(core_axis_name="c", subcore_axis_name="s"),
    out_type=jax.ShapeDtypeStruct((NPAD, HID), jnp.float32),
    scratch_types=[
        pltpu.VMEM((RPT_LEN,), jnp.int32),
        pltpu.VMEM((CH,), jnp.int32),
        pltpu.VMEM((CH, HID), jnp.float32),
        pltpu.VMEM((NPW, HID), jnp.float32),
        pltpu.VMEM((FB, HID), jnp.float32),
        pltpu.SemaphoreType.DMA,
        pltpu.SemaphoreType.DMA,
    ],
)
def _sc_kernel(p_hbm, s_hbm, idx_hbm, rpt_hbm, g_hbm,
               rpt_v, idx_v, rows_v, s_all, flush_v, sem, sem2):
    _sc_segment_gelu(p_hbm, s_hbm, idx_hbm, rpt_hbm, g_hbm,
                     rpt_v, idx_v, rows_v, s_all, flush_v, sem, sem2)


def kernel(in_features, W1, b1, W2, b2, neighbor_idx, rowptr):
    x = in_features[0]
    xp = jnp.pad(x, ((0, NPAD - N), (0, 0)))
    wc = jnp.concatenate([W1[:C_IN], W1[C_IN:]], axis=1)
    b1r = b1.reshape(1, HID)

    nblk = NPAD // 512
    p_arr, s_arr = pl.pallas_call(
        _mm_ps_body,
        grid=(nblk,),
        in_specs=[
            pl.BlockSpec((512, C_IN), lambda i: (i, 0)),
            pl.BlockSpec((C_IN, 2 * HID), lambda i: (0, 0)),
            pl.BlockSpec((1, HID), lambda i: (0, 0)),
        ],
        out_specs=[
            pl.BlockSpec((512, HID), lambda i: (i, 0)),
            pl.BlockSpec((512, HID), lambda i: (i, 0)),
        ],
        out_shape=[
            jax.ShapeDtypeStruct((NPAD, HID), jnp.float32),
            jax.ShapeDtypeStruct((NPAD, HID), jnp.float32),
        ],
    )(xp, wc, b1r)

    idx32 = neighbor_idx.astype(jnp.int32)
    rpt32 = rowptr.astype(jnp.int32)
    idxp = jnp.pad(idx32, (0, EPAD - E))
    rptp = jnp.pad(rpt32, (0, RPT_PAD - (N + 1)), constant_values=E)

    g_arr = _sc_kernel(p_arr, s_arr, idxp, rptp)

    rhi = rptp[1:NPAD + 1].reshape(nblk, 1, 512)
    rlo = rptp[:NPAD].reshape(nblk, 1, 512)
    b2r = b2.reshape(1, C_OUT)

    out = pl.pallas_call(
        _mm_out_body,
        grid=(nblk,),
        in_specs=[
            pl.BlockSpec((512, HID), lambda i: (i, 0)),
            pl.BlockSpec((HID, C_OUT), lambda i: (0, 0)),
            pl.BlockSpec((1, C_OUT), lambda i: (0, 0)),
            pl.BlockSpec((1, 1, 512), lambda i: (i, 0, 0)),
            pl.BlockSpec((1, 1, 512), lambda i: (i, 0, 0)),
        ],
        out_specs=pl.BlockSpec((512, C_OUT), lambda i: (i, 0)),
        out_shape=jax.ShapeDtypeStruct((NPAD, C_OUT), jnp.float32),
    )(g_arr, W2, b2r, rhi, rlo)

    return out[:N].reshape(1, N, C_OUT)


# speedup vs baseline: 1.7176x; 1.7176x over previous
import functools

import jax
import jax.numpy as jnp
from jax import lax
from jax.experimental import pallas as pl
from jax.experimental.pallas import tpu as pltpu
from jax.experimental.pallas import tpu_sc as plsc

N = 10000
E = 320000
C_IN = 128
HID = 256
C_OUT = 128

NC = 2
NS = 16
NW = NC * NS

NPW = 320
NPAD = NW * NPW
RPT_LEN = NPW + 24
RPT_PAD = (NW - 1) * NPW + RPT_LEN
CH = 128
FB = 16
EPAD = E + CH

GA = -2.0 * 0.7978845608028654
GB = GA * 0.044715

VB = HID // 16


def _mm_ps_body(x_ref, w_ref, b1_ref, p_ref, s_ref):
    ps = jnp.dot(x_ref[...], w_ref[...], preferred_element_type=jnp.float32)
    p_ref[...] = ps[:, :HID]
    s_ref[...] = ps[:, HID:] + b1_ref[...]


def _mm_out_body(g_ref, w2_ref, b2_ref, rhi_ref, rlo_ref, o_ref):
    cnt = (rhi_ref[0, 0, :] - rlo_ref[0, 0, :]).astype(jnp.float32)
    scale = 1.0 / jnp.maximum(cnt, 1.0)
    gs = g_ref[...] * scale[:, None]
    y = jnp.dot(gs, w2_ref[...], preferred_element_type=jnp.float32)
    o_ref[...] = y + b2_ref[...] * (cnt > 0.0).astype(jnp.float32)[:, None]


def _sc_segment_gelu(p_hbm, s_hbm, idx_hbm, rpt_hbm, g_hbm,
                     rpt_v, idx_v, rows_v, s_all, flush_v, sem, sem2):
    c = lax.axis_index("c")
    s = lax.axis_index("s")
    wid = s * NC + c
    n0 = wid * NPW

    pltpu.async_copy(rpt_hbm.at[pl.ds(pl.multiple_of(n0, 8), RPT_LEN)],
                     rpt_v, sem2).wait()
    pltpu.async_copy(s_hbm.at[pl.ds(pl.multiple_of(n0, 8), NPW)],
                     s_all, sem2).wait()

    def rv(k):
        return rpt_v[pl.ds(k, 16)][0]

    zeros16 = jnp.zeros((16,), jnp.float32)

    def load_chunk(a):
        pltpu.async_copy(idx_hbm.at[pl.ds(pl.multiple_of(a, 8), CH)],
                         idx_v, sem2).wait()
        pltpu.async_copy(p_hbm.at[idx_v], rows_v, sem).wait()

    e_start = rv(0)
    e_end = rv(NPW)
    a_init = (e_start // 8) * 8
    load_chunk(a_init)
    nchunks = (e_end - a_init + CH - 1) // CH
    n_events = NPW + jnp.maximum(nchunks - 1, 0)

    def event_body(_, st):
        i, a, p, acc = st
        e1 = rv(i + 1)
        chunk_end = a + CH
        hi = jnp.minimum(e1, chunk_end)
        srow = tuple(s_all[i, pl.ds(j * 16, 16)] for j in range(VB))

        def edge_body(r, acc_):
            new_acc = []
            for j in range(VB):
                x = rows_v[r, pl.ds(j * 16, 16)] + srow[j]
                z = x * (GA + GB * (x * x))
                new_acc.append(acc_[j] + x / (1.0 + jnp.exp(z)))
            return tuple(new_acc)

        acc = lax.fori_loop(p - a, hi - a, edge_body, acc)
        node_done = jnp.logical_and(e1 <= chunk_end, i < NPW)

        @pl.when(node_done)
        def _():
            for j in range(VB):
                flush_v[i % FB, pl.ds(j * 16, 16)] = acc[j]

        @pl.when(jnp.logical_and(node_done, i % FB == FB - 1))
        def _():
            pltpu.async_copy(
                flush_v,
                g_hbm.at[pl.ds(pl.multiple_of(n0 + (i // FB) * FB, 8), FB)],
                sem2).wait()

        a_next = jnp.minimum(a + CH, E)

        @pl.when(jnp.logical_not(node_done))
        def _():
            load_chunk(a_next)

        keep = jnp.where(node_done, 0.0, 1.0).astype(jnp.float32)
        acc = tuple(acc[j] * keep for j in range(VB))
        i = i + node_done.astype(jnp.int32)
        a = jnp.where(node_done, a, a_next)
        return (i, a, hi, acc)

    lax.fori_loop(0, n_events, event_body,
                  (0, a_init, e_start, (zeros16,) * VB))


@functools.partial(
    pl.kernel,
    mesh=plsc.VectorSubcoreMesh(core_axis_name="c", subcore_axis_name="s"),
    out_type=jax.ShapeDtypeStruct((NPAD, HID), jnp.float32),
    scratch_types=[
        pltpu.VMEM((RPT_LEN,), jnp.int32),
        pltpu.VMEM((CH,), jnp.int32),
        pltpu.VMEM((CH, HID), jnp.float32),
        pltpu.VMEM((NPW, HID), jnp.float32),
        pltpu.VMEM((FB, HID), jnp.float32),
        pltpu.SemaphoreType.DMA,
        pltpu.SemaphoreType.DMA,
    ],
)
def _sc_kernel(p_hbm, s_hbm, idx_hbm, rpt_hbm, g_hbm,
               rpt_v, idx_v, rows_v, s_all, flush_v, sem, sem2):
    _sc_segment_gelu(p_hbm, s_hbm, idx_hbm, rpt_hbm, g_hbm,
                     rpt_v, idx_v, rows_v, s_all, flush_v, sem, sem2)


def kernel(in_features, W1, b1, W2, b2, neighbor_idx, rowptr):
    x = in_features[0]
    xp = jnp.pad(x, ((0, NPAD - N), (0, 0)))
    wc = jnp.concatenate([W1[:C_IN], W1[C_IN:]], axis=1)
    b1r = b1.reshape(1, HID)

    nblk = NPAD // 512
    p_arr, s_arr = pl.pallas_call(
        _mm_ps_body,
        grid=(nblk,),
        in_specs=[
            pl.BlockSpec((512, C_IN), lambda i: (i, 0)),
            pl.BlockSpec((C_IN, 2 * HID), lambda i: (0, 0)),
            pl.BlockSpec((1, HID), lambda i: (0, 0)),
        ],
        out_specs=[
            pl.BlockSpec((512, HID), lambda i: (i, 0)),
            pl.BlockSpec((512, HID), lambda i: (i, 0)),
        ],
        out_shape=[
            jax.ShapeDtypeStruct((NPAD, HID), jnp.float32),
            jax.ShapeDtypeStruct((NPAD, HID), jnp.float32),
        ],
    )(xp, wc, b1r)

    idx32 = neighbor_idx.astype(jnp.int32)
    rpt32 = rowptr.astype(jnp.int32)
    idxp = jnp.pad(idx32, (0, EPAD - E))
    rptp = jnp.pad(rpt32, (0, RPT_PAD - (N + 1)), constant_values=E)

    g_arr = _sc_kernel(p_arr, s_arr, idxp, rptp)

    rhi = rptp[1:NPAD + 1].reshape(nblk, 1, 512)
    rlo = rptp[:NPAD].reshape(nblk, 1, 512)
    b2r = b2.reshape(1, C_OUT)

    out = pl.pallas_call(
        _mm_out_body,
        grid=(nblk,),
        in_specs=[
            pl.BlockSpec((512, HID), lambda i: (i, 0)),
            pl.BlockSpec((HID, C_OUT), lambda i: (0, 0)),
            pl.BlockSpec((1, C_OUT), lambda i: (0, 0)),
            pl.BlockSpec((1, 1, 512), lambda i: (i, 0, 0)),
            pl.BlockSpec((1, 1, 512), lambda i: (i, 0, 0)),
        ],
        out_specs=pl.BlockSpec((512, C_OUT), lambda i: (i, 0)),
        out_shape=jax.ShapeDtypeStruct((NPAD, C_OUT), jnp.float32),
    )(g_arr, W2, b2r, rhi, rlo)

    return out[:N].reshape(1, N, C_OUT)


# speedup vs baseline: 2.3866x; 1.3895x over previous
import functools

import jax
import jax.numpy as jnp
from jax import lax
from jax.experimental import pallas as pl
from jax.experimental.pallas import tpu as pltpu
from jax.experimental.pallas import tpu_sc as plsc

N = 10000
E = 320000
C_IN = 128
HID = 256
C_OUT = 128

NC = 2
NS = 16
NW = NC * NS

NPW = 320
NPAD = NW * NPW
RPT_LEN = NPW + 24
RPT_PAD = (NW - 1) * NPW + RPT_LEN
CH = 64
FB = 16
EPAD = E + CH

GA = -2.0 * 0.7978845608028654
GB = GA * 0.044715

VB = HID // 16


def _mm_ps_body(x_ref, w_ref, b1_ref, p_ref, s_ref):
    ps = jnp.dot(x_ref[...], w_ref[...], preferred_element_type=jnp.float32)
    p_ref[...] = ps[:, :HID]
    s_ref[...] = ps[:, HID:] + b1_ref[...]


def _mm_out_body(g_ref, w2_ref, b2_ref, rhi_ref, rlo_ref, o_ref):
    cnt = (rhi_ref[0, 0, :] - rlo_ref[0, 0, :]).astype(jnp.float32)
    scale = 1.0 / jnp.maximum(cnt, 1.0)
    gs = g_ref[...] * scale[:, None]
    y = jnp.dot(gs, w2_ref[...], preferred_element_type=jnp.float32)
    o_ref[...] = y + b2_ref[...] * (cnt > 0.0).astype(jnp.float32)[:, None]


def _sc_segment_gelu(p_hbm, s_hbm, idx_hbm, rpt_hbm, g_hbm,
                     rpt_v, idx2_v, rows2_v, s_all, flush_v,
                     semi, semg, sem2):
    c = lax.axis_index("c")
    s = lax.axis_index("s")
    wid = s * NC + c
    n0 = wid * NPW

    pltpu.async_copy(rpt_hbm.at[pl.ds(pl.multiple_of(n0, 8), RPT_LEN)],
                     rpt_v, sem2).wait()
    pltpu.async_copy(s_hbm.at[pl.ds(pl.multiple_of(n0, 8), NPW)],
                     s_all, sem2).wait()

    def rv(k):
        return rpt_v[pl.ds(k, 16)][0]

    zeros16 = jnp.zeros((16,), jnp.float32)

    def issue_idx(a, slot):
        pltpu.async_copy(
            idx_hbm.at[pl.ds(pl.multiple_of(jnp.minimum(a, E), 8), CH)],
            idx2_v.at[slot], semi)

    def issue_gather(slot):
        pltpu.async_copy(p_hbm.at[idx2_v.at[slot]], rows2_v.at[slot], semg)

    def drain_gather():
        pltpu.make_async_copy(p_hbm.at[pl.ds(0, CH)], rows2_v.at[0],
                              semg).wait()

    def drain_idx():
        pltpu.make_async_copy(idx_hbm.at[pl.ds(0, CH)], idx2_v.at[0],
                              semi).wait()

    e_start = rv(0)
    e_end = rv(NPW)
    a_init = (e_start // 8) * 8
    nchunks = (e_end - a_init + CH - 1) // CH
    n_events = NPW + jnp.maximum(nchunks - 1, 0)

    issue_idx(a_init, 0)
    drain_idx()
    issue_gather(0)
    issue_idx(a_init + CH, 1)
    drain_gather()
    drain_idx()
    issue_gather(1)
    issue_idx(a_init + 2 * CH, 0)

    def event_body(_, st):
        i, a, p, par, acc = st
        e1 = rv(i + 1)
        chunk_end = a + CH
        hi = jnp.minimum(e1, chunk_end)
        srow = tuple(s_all[i, pl.ds(j * 16, 16)] for j in range(VB))

        def edge_body(r, acc_):
            new_acc = []
            for j in range(VB):
                x = rows2_v[par, r, pl.ds(j * 16, 16)] + srow[j]
                z = x * (GA + GB * (x * x))
                new_acc.append(acc_[j] + x / (1.0 + jnp.exp(z)))
            return tuple(new_acc)

        acc = lax.fori_loop(p - a, hi - a, edge_body, acc)
        node_done = jnp.logical_and(e1 <= chunk_end, i < NPW)

        @pl.when(node_done)
        def _():
            for j in range(VB):
                flush_v[i % FB, pl.ds(j * 16, 16)] = acc[j]

        @pl.when(jnp.logical_and(node_done, i % FB == FB - 1))
        def _():
            pltpu.async_copy(
                flush_v,
                g_hbm.at[pl.ds(pl.multiple_of(n0 + (i // FB) * FB, 8), FB)],
                sem2).wait()

        npar = 1 - par

        @pl.when(jnp.logical_not(node_done))
        def _():
            drain_gather()
            drain_idx()
            issue_gather(par)
            issue_idx(a + 3 * CH, npar)

        keep = jnp.where(node_done, 0.0, 1.0).astype(jnp.float32)
        acc = tuple(acc[j] * keep for j in range(VB))
        i = i + node_done.astype(jnp.int32)
        a = jnp.where(node_done, a, a + CH)
        par = jnp.where(node_done, par, npar)
        return (i, a, hi, par, acc)

    lax.fori_loop(0, n_events, event_body,
                  (0, a_init, e_start, 0, (zeros16,) * VB))

    drain_gather()
    drain_idx()


@functools.partial(
    pl.kernel,
    mesh=plsc.VectorSubcoreMesh(core_axis_name="c", subcore_axis_name="s"),
    out_type=jax.ShapeDtypeStruct((NPAD, HID), jnp.float32),
    scratch_types=[
        pltpu.VMEM((RPT_LEN,), jnp.int32),
        pltpu.VMEM((2, CH), jnp.int32),
        pltpu.VMEM((2, CH, HID), jnp.float32),
        pltpu.VMEM((NPW, HID), jnp.float32),
        pltpu.VMEM((FB, HID), jnp.float32),
        pltpu.SemaphoreType.DMA,
        pltpu.SemaphoreType.DMA,
        pltpu.SemaphoreType.DMA,
    ],
)
def _sc_kernel(p_hbm, s_hbm, idx_hbm, rpt_hbm, g_hbm,
               rpt_v, idx2_v, rows2_v, s_all, flush_v, semi, semg, sem2):
    _sc_segment_gelu(p_hbm, s_hbm, idx_hbm, rpt_hbm, g_hbm,
                     rpt_v, idx2_v, rows2_v, s_all, flush_v,
                     semi, semg, sem2)


def kernel(in_features, W1, b1, W2, b2, neighbor_idx, rowptr):
    x = in_features[0]
    xp = jnp.pad(x, ((0, NPAD - N), (0, 0)))
    wc = jnp.concatenate([W1[:C_IN], W1[C_IN:]], axis=1)
    b1r = b1.reshape(1, HID)

    nblk = NPAD // 512
    p_arr, s_arr = pl.pallas_call(
        _mm_ps_body,
        grid=(nblk,),
        in_specs=[
            pl.BlockSpec((512, C_IN), lambda i: (i, 0)),
            pl.BlockSpec((C_IN, 2 * HID), lambda i: (0, 0)),
            pl.BlockSpec((1, HID), lambda i: (0, 0)),
        ],
        out_specs=[
            pl.BlockSpec((512, HID), lambda i: (i, 0)),
            pl.BlockSpec((512, HID), lambda i: (i, 0)),
        ],
        out_shape=[
            jax.ShapeDtypeStruct((NPAD, HID), jnp.float32),
            jax.ShapeDtypeStruct((NPAD, HID), jnp.float32),
        ],
    )(xp, wc, b1r)

    idx32 = neighbor_idx.astype(jnp.int32)
    rpt32 = rowptr.astype(jnp.int32)
    idxp = jnp.pad(idx32, (0, EPAD - E))
    rptp = jnp.pad(rpt32, (0, RPT_PAD - (N + 1)), constant_values=E)

    g_arr = _sc_kernel(p_arr, s_arr, idxp, rptp)

    rhi = rptp[1:NPAD + 1].reshape(nblk, 1, 512)
    rlo = rptp[:NPAD].reshape(nblk, 1, 512)
    b2r = b2.reshape(1, C_OUT)

    out = pl.pallas_call(
        _mm_out_body,
        grid=(nblk,),
        in_specs=[
            pl.BlockSpec((512, HID), lambda i: (i, 0)),
            pl.BlockSpec((HID, C_OUT), lambda i: (0, 0)),
            pl.BlockSpec((1, C_OUT), lambda i: (0, 0)),
            pl.BlockSpec((1, 1, 512), lambda i: (i, 0, 0)),
            pl.BlockSpec((1, 1, 512), lambda i: (i, 0, 0)),
        ],
        out_specs=pl.BlockSpec((512, C_OUT), lambda i: (i, 0)),
        out_shape=jax.ShapeDtypeStruct((NPAD, C_OUT), jnp.float32),
    )(g_arr, W2, b2r, rhi, rlo)

    return out[:N].reshape(1, N, C_OUT)
